# baseline (speedup 1.0000x reference)
import jax
import jax.numpy as jnp
from jax import lax
from jax.experimental import pallas as pl
from jax.experimental.pallas import tpu as pltpu
from jax.experimental.pallas import tpu_sc as plsc

_B = 16384
_C = 5
_D = 64
_NC = 2
_NS = 16
_NW = _NC * _NS
_BPW = _B // _NW
_CB = 128
_NCHUNK = _BPW // _CB
_PAIRS = _CB * _C
_W = 2 * _D


def _sc_body(tgt_hbm, ctx_hbm, bmap_hbm, t128_hbm, c128_hbm, out_hbm,
             tidx, cidx, tidx_hi, cidx_hi, bmapv, trows, crows, outbuf, sem):
    wid = lax.axis_index("s") * _NC + lax.axis_index("c")
    base = wid * _BPW
    pltpu.sync_copy(bmap_hbm, bmapv)
    iota = lax.iota(jnp.int32, 16)
    for chunk in range(_NCHUNK):
        b0 = base + chunk * _CB
        pltpu.sync_copy(tgt_hbm.at[pl.ds(b0, _CB)], tidx)
        pltpu.sync_copy(ctx_hbm.at[pl.ds(b0 * _C, _PAIRS)], cidx)

        def prep_t(m, carry):
            tidx_hi[pl.ds(m * 16, 16)] = (
                lax.shift_right_logical(tidx[pl.ds(m * 16, 16)], 1))
            return carry

        lax.fori_loop(0, _CB // 16, prep_t, 0)

        def prep_c(m, carry):
            v = lax.shift_right_logical(cidx[pl.ds(m * 16, 16)], 1)
            cidx_hi[lax.shift_right_logical(m, 3),
                    pl.ds((m % 8) * 16, 16)] = v
            return carry

        lax.fori_loop(0, _PAIRS // 16, prep_c, 0)

        cp_t = pltpu.make_async_copy(t128_hbm.at[tidx_hi], trows, sem)
        cp_t.start()
        cps = []
        for j in range(_C):
            cp = pltpu.make_async_copy(c128_hbm.at[cidx_hi.at[j]],
                                       crows.at[pl.ds(j * _CB, _CB)], sem)
            cp.start()
            cps.append(cp)
        cp_t.wait()
        for cp in cps:
            cp.wait()

        def body(g, carry):
            p0 = g * 16
            b_l = bmapv[pl.ds(p0, 16)]
            craw = cidx[pl.ds(p0, 16)]
            traw = plsc.load_gather(tidx, [b_l])
            wbase = b_l * 128 + (traw & 1) * 64
            cbase = (p0 + iota) * 128 + (craw & 1) * 64
            acc = jnp.zeros((16,), jnp.float32)
            for e in range(_D):
                we = wbase + e
                ce = cbase + e
                wv = plsc.load_gather(trows, [lax.shift_right_logical(we, 7),
                                              we & 127])
                cv = plsc.load_gather(crows, [lax.shift_right_logical(ce, 7),
                                              ce & 127])
                acc = acc + wv * cv
            outbuf[pl.ds(p0, 16)] = acc
            return carry

        lax.fori_loop(0, _PAIRS // 16, body, 0)
        pltpu.sync_copy(outbuf, out_hbm.at[pl.ds(b0 * _C, _PAIRS)])


@jax.jit
def kernel(target, context, target_table, context_table):
    tgt = target.astype(jnp.int32)
    ctx = context.reshape(-1).astype(jnp.int32)
    bmap = (jnp.arange(_PAIRS, dtype=jnp.int32) // _C)
    t128 = jnp.reshape(target_table, (500000, _W))
    c128 = jnp.reshape(context_table, (500000, _W))
    mesh = plsc.VectorSubcoreMesh(core_axis_name="c", subcore_axis_name="s",
                                  num_cores=_NC, num_subcores=_NS)
    out_flat = pl.kernel(
        _sc_body,
        out_type=jax.ShapeDtypeStruct((_B * _C,), jnp.float32),
        mesh=mesh,
        compiler_params=pltpu.CompilerParams(needs_layout_passes=False,
                                             use_tc_tiling_on_sc=True),
        scratch_types=[
            pltpu.VMEM((_CB,), jnp.int32),
            pltpu.VMEM((_PAIRS,), jnp.int32),
            pltpu.VMEM((_CB,), jnp.int32),
            pltpu.VMEM((_C, _CB), jnp.int32),
            pltpu.VMEM((_PAIRS,), jnp.int32),
            pltpu.VMEM((_CB, _W), jnp.float32),
            pltpu.VMEM((_PAIRS, _W), jnp.float32),
            pltpu.VMEM((_PAIRS,), jnp.float32),
            pltpu.SemaphoreType.DMA,
        ],
    )(tgt, ctx, bmap, t128, c128)
    return out_flat.reshape(_B, _C)
